# baseline (speedup 1.0000x reference)
import functools

import jax
import jax.numpy as jnp
from jax import lax
from jax.experimental import pallas as pl
from jax.experimental.pallas import tpu as pltpu
from jax.experimental.pallas import tpu_sc as plsc

_N = 10000
_D = 128
_E = 320000

_NC = 2
_NS = 16
_NW = _NC * _NS

_K = 128
_NB = 79
_EPW = _K * _NB
_EPAD = 2 * _EPW * _NS

_NB0 = 52
_NB1 = 2 * _NB - _NB0
_CD = 8


def _rup8(n):
    return (n + 7) // 8 * 8

_RPT = 632
_N1 = _RPT * _NS

_mesh = plsc.VectorSubcoreMesh(core_axis_name="c", subcore_axis_name="s")


@functools.partial(
    pl.kernel,
    out_type=jax.ShapeDtypeStruct((_NC, _N1, _D), jnp.float32),
    mesh=_mesh,
    scratch_types=[
        pltpu.VMEM((_NB, _K), jnp.int32),
        pltpu.VMEM((_K, _D), jnp.float32),
        pltpu.VMEM_SHARED((_N1, _D), jnp.float32),
    ],
)
def _deg_kernel(dst_hbm, ones_hbm, zeros_hbm, out_hbm, dst_v, ones_v, deg_sp):
    c = lax.axis_index("c")
    s = lax.axis_index("s")
    wid = s * _NC + c
    r0 = s * _RPT
    pltpu.sync_copy(zeros_hbm.at[pl.ds(r0, _RPT)], deg_sp.at[pl.ds(r0, _RPT)])
    pltpu.sync_copy(ones_hbm, ones_v)
    pltpu.sync_copy(dst_hbm.at[wid], dst_v)
    plsc.subcore_barrier()

    def body(j, carry):
        pltpu.sync_copy(ones_v, deg_sp.at[dst_v.at[j]], add=True)
        return carry

    lax.fori_loop(0, _NB, body, 0)
    plsc.subcore_barrier()
    pltpu.sync_copy(deg_sp.at[pl.ds(r0, _RPT)], out_hbm.at[c, pl.ds(r0, _RPT)])


@functools.partial(
    pl.kernel,
    out_type=jax.ShapeDtypeStruct((_NC, _N1, _D), jnp.float32),
    mesh=_mesh,
    scratch_types=[
        pltpu.VMEM((2, _CD, _K), jnp.int32),
        pltpu.VMEM((2, _CD, _K), jnp.int32),
        pltpu.VMEM((2, _D, _D), jnp.float32),
        pltpu.VMEM_SHARED((_N1, _D), jnp.float32),
        pltpu.SemaphoreType.DMA,
        pltpu.SemaphoreType.DMA,
        pltpu.SemaphoreType.DMA,
        pltpu.SemaphoreType.DMA,
        pltpu.SemaphoreType.DMA,
        pltpu.SemaphoreType.DMA,
        pltpu.SemaphoreType.DMA,
        pltpu.SemaphoreType.DMA,
    ],
)
def _scatter_kernel(srcA, dstA, srcB, dstB, g_hbm, zeros_hbm, out_hbm,
                    srcb, dstb, rows, acc_sp,
                    gsem0, gsem1, ssem0, ssem1, csem0, csem1, xsem0, xsem1):
    c = lax.axis_index("c")
    s = lax.axis_index("s")
    r0 = s * _RPT
    pltpu.sync_copy(zeros_hbm.at[pl.ds(r0, _RPT)], acc_sp.at[pl.ds(r0, _RPT)])
    plsc.subcore_barrier()

    gsem = (gsem0, gsem1)
    ssem = (ssem0, ssem1)
    csem = (csem0, csem1)
    xsem = (xsem0, xsem1)

    def consume(p, q, i, pq, pi):
        pltpu.make_async_copy(g_hbm.at[srcb.at[0, 0]], rows.at[p], gsem[p]).wait()
        pltpu.async_copy(rows.at[p], acc_sp.at[dstb.at[q, i]], ssem[p], add=True)
        pltpu.make_async_copy(rows.at[p], acc_sp.at[dstb.at[q, i]], ssem[p]).wait()
        if pq is not None:
            pltpu.async_copy(g_hbm.at[srcb.at[pq, pi]], rows.at[p], gsem[p])

    def run(src_arr, dst_arr, nb):
        nch = (nb + _CD - 1) // _CD
        pltpu.sync_copy(src_arr.at[s, pl.ds(0, _CD)], srcb.at[0])
        pltpu.sync_copy(dst_arr.at[s, pl.ds(0, _CD)], dstb.at[0])
        if nch > 1:
            pltpu.async_copy(src_arr.at[s, pl.ds(_CD, _CD)], srcb.at[1], xsem1)
            pltpu.async_copy(dst_arr.at[s, pl.ds(_CD, _CD)], dstb.at[1], csem1)
        pltpu.async_copy(g_hbm.at[srcb.at[0, 0]], rows.at[0], gsem0)
        pltpu.async_copy(g_hbm.at[srcb.at[0, 1]], rows.at[1], gsem1)

        for ch in range(nch):
            q = ch % 2
            base = ch * _CD
            last = ch + 1 == nch
            if ch > 0:
                pltpu.make_async_copy(
                    dst_arr.at[s, pl.ds(base, _CD)], dstb.at[q], csem[q]).wait()
                if not last:
                    pltpu.async_copy(
                        src_arr.at[s, pl.ds(base + _CD, _CD)],
                        srcb.at[1 - q], xsem[1 - q])
                    pltpu.async_copy(
                        dst_arr.at[s, pl.ds(base + _CD, _CD)],
                        dstb.at[1 - q], csem[1 - q])

            rb = min(_CD, nb - base)
            if not last:
                def body(t, carry, q=q):
                    consume(0, q, 2 * t, q, 2 * t + 2)
                    consume(1, q, 2 * t + 1, q, 2 * t + 3)
                    return carry
                lax.fori_loop(0, _CD // 2 - 1, body, 0)
                pltpu.make_async_copy(
                    src_arr.at[s, pl.ds(base + _CD, _CD)],
                    srcb.at[1 - q], xsem[1 - q]).wait()
                consume(0, q, _CD - 2, 1 - q, 0)
                consume(1, q, _CD - 1, 1 - q, 1)
            else:
                def tbody(t, carry, q=q):
                    consume(0, q, 2 * t, q, 2 * t + 2)
                    consume(1, q, 2 * t + 1, q, 2 * t + 3)
                    return carry
                lax.fori_loop(0, (rb - 2) // 2, tbody, 0)
                consume(0, q, rb - 2, None, None)
                consume(1, q, rb - 1, None, None)

    @pl.when(c == 0)
    def _():
        run(srcA, dstA, _NB0)

    @pl.when(c == 1)
    def _():
        run(srcB, dstB, _NB1)

    plsc.subcore_barrier()
    pltpu.sync_copy(acc_sp.at[pl.ds(r0, _RPT)], out_hbm.at[c, pl.ds(r0, _RPT)])


def _gemm_block(x_ref, w_ref, dp_ref, g_ref):
    deg = dp_ref[0, :, 0] + dp_ref[1, :, 0] + 1.0
    dis = lax.rsqrt(deg)
    h = jnp.dot(x_ref[...], w_ref[...], preferred_element_type=jnp.float32)
    g_ref[...] = h * dis[:, None]


def _combine_block(p_ref, g_ref, dp_ref, b_ref, o_ref):
    deg = dp_ref[0, :, 0] + dp_ref[1, :, 0] + 1.0
    dis = lax.rsqrt(deg)
    t = (p_ref[0] + p_ref[1] + g_ref[...]) * dis[:, None] + b_ref[...]
    o_ref[...] = jnp.maximum(t, 0.0)


_RB = 1024


def kernel(x, edge_index, W, b):
    src = edge_index[0]
    dst = edge_index[1]
    pad = _EPAD - _E
    src_p = jnp.concatenate([src, jnp.zeros((pad,), jnp.int32)])
    dst_p = jnp.concatenate([dst, jnp.full((pad,), _N, jnp.int32)])
    dst_r = dst_p.reshape(_NW, _NB, _K)
    sp2 = src_p.reshape(_NS, 2 * _EPW)
    dp2 = dst_p.reshape(_NS, 2 * _EPW)
    cut = _NB0 * _K
    padA = _rup8(_NB0) - _NB0
    padB = _rup8(_NB1) - _NB1
    srcA = jnp.concatenate(
        [sp2[:, :cut].reshape(_NS, _NB0, _K),
         jnp.zeros((_NS, padA, _K), jnp.int32)], axis=1)
    dstA = jnp.concatenate(
        [dp2[:, :cut].reshape(_NS, _NB0, _K),
         jnp.full((_NS, padA, _K), _N, jnp.int32)], axis=1)
    srcB = jnp.concatenate(
        [sp2[:, cut:].reshape(_NS, _NB1, _K),
         jnp.zeros((_NS, padB, _K), jnp.int32)], axis=1)
    dstB = jnp.concatenate(
        [dp2[:, cut:].reshape(_NS, _NB1, _K),
         jnp.full((_NS, padB, _K), _N, jnp.int32)], axis=1)

    zerosd = jnp.zeros((_N1, _D), jnp.float32)
    onesd = jnp.ones((_K, _D), jnp.float32)

    deg_parts = _deg_kernel(dst_r, onesd, zerosd)

    g = pl.pallas_call(
        _gemm_block,
        grid=((_N + _RB - 1) // _RB,),
        in_specs=[
            pl.BlockSpec((_RB, _D), lambda i: (i, 0)),
            pl.BlockSpec((_D, _D), lambda i: (0, 0)),
            pl.BlockSpec((_NC, _RB, _D), lambda i: (0, i, 0)),
        ],
        out_specs=pl.BlockSpec((_RB, _D), lambda i: (i, 0)),
        out_shape=jax.ShapeDtypeStruct((_N, _D), jnp.float32),
    )(x, W, deg_parts)

    acc_parts = _scatter_kernel(srcA, dstA, srcB, dstB, g, zerosd)

    out = pl.pallas_call(
        _combine_block,
        grid=((_N + _RB - 1) // _RB,),
        in_specs=[
            pl.BlockSpec((_NC, _RB, _D), lambda i: (0, i, 0)),
            pl.BlockSpec((_RB, _D), lambda i: (i, 0)),
            pl.BlockSpec((_NC, _RB, _D), lambda i: (0, i, 0)),
            pl.BlockSpec((1, _D), lambda i: (0, 0)),
        ],
        out_specs=pl.BlockSpec((_RB, _D), lambda i: (i, 0)),
        out_shape=jax.ShapeDtypeStruct((_N, _D), jnp.float32),
    )(acc_parts, g, deg_parts, b.reshape(1, _D))

    return out


# speedup vs baseline: 1.0000x; 1.0000x over previous
import functools

import jax
import jax.numpy as jnp
from jax import lax
from jax.experimental import pallas as pl
from jax.experimental.pallas import tpu as pltpu
from jax.experimental.pallas import tpu_sc as plsc

_N = 10000
_D = 128
_E = 320000

_NC = 2
_NS = 16
_NW = _NC * _NS

_K = 128
_NB = 79
_EPW = _K * _NB
_EPAD = 2 * _EPW * _NS

_NB0 = 52
_NB1 = 2 * _NB - _NB0
_CD = 8


def _rup8(n):
    return (n + 7) // 8 * 8

_RPT = 632
_N1 = _RPT * _NS

_mesh = plsc.VectorSubcoreMesh(core_axis_name="c", subcore_axis_name="s")


@functools.partial(
    pl.kernel,
    out_type=jax.ShapeDtypeStruct((_NC, _N1, _D), jnp.float32),
    mesh=_mesh,
    scratch_types=[
        pltpu.VMEM((_NB, _K), jnp.int32),
        pltpu.VMEM((_K, _D), jnp.float32),
        pltpu.VMEM_SHARED((_N1, _D), jnp.float32),
    ],
)
def _deg_kernel(dst_hbm, ones_hbm, zeros_hbm, out_hbm, dst_v, ones_v, deg_sp):
    c = lax.axis_index("c")
    s = lax.axis_index("s")
    wid = s * _NC + c
    r0 = s * _RPT
    pltpu.sync_copy(zeros_hbm.at[pl.ds(r0, _RPT)], deg_sp.at[pl.ds(r0, _RPT)])
    pltpu.sync_copy(ones_hbm, ones_v)
    pltpu.sync_copy(dst_hbm.at[wid], dst_v)
    plsc.subcore_barrier()

    def body(j, carry):
        pltpu.sync_copy(ones_v, deg_sp.at[dst_v.at[j]], add=True)
        return carry

    lax.fori_loop(0, _NB, body, 0)
    plsc.subcore_barrier()
    pltpu.sync_copy(deg_sp.at[pl.ds(r0, _RPT)], out_hbm.at[c, pl.ds(r0, _RPT)])


@functools.partial(
    pl.kernel,
    out_type=jax.ShapeDtypeStruct((_NC, _N1, _D), jnp.float32),
    mesh=_mesh,
    scratch_types=[
        pltpu.VMEM((112, _K), jnp.int32),
        pltpu.VMEM((2, _CD, _K), jnp.int32),
        pltpu.VMEM((2, _D, _D), jnp.float32),
        pltpu.VMEM_SHARED((_N1, _D), jnp.float32),
        pltpu.SemaphoreType.DMA,
        pltpu.SemaphoreType.DMA,
        pltpu.SemaphoreType.DMA,
        pltpu.SemaphoreType.DMA,
        pltpu.SemaphoreType.DMA,
        pltpu.SemaphoreType.DMA,
    ],
)
def _scatter_kernel(srcA, dstA, srcB, dstB, g_hbm, zeros_hbm, out_hbm,
                    src_v, dstb, rows, acc_sp,
                    gsem0, gsem1, ssem0, ssem1, csem0, csem1):
    c = lax.axis_index("c")
    s = lax.axis_index("s")
    r0 = s * _RPT
    pltpu.sync_copy(zeros_hbm.at[pl.ds(r0, _RPT)], acc_sp.at[pl.ds(r0, _RPT)])
    plsc.subcore_barrier()

    gsem = (gsem0, gsem1)
    ssem = (ssem0, ssem1)
    csem = (csem0, csem1)

    def consume(j, p, q, i, gather_next):
        pltpu.make_async_copy(g_hbm.at[src_v.at[j]], rows.at[p], gsem[p]).wait()
        pltpu.async_copy(rows.at[p], acc_sp.at[dstb.at[q, i]], ssem[p], add=True)
        pltpu.make_async_copy(rows.at[p], acc_sp.at[dstb.at[q, i]], ssem[p]).wait()
        if gather_next:
            pltpu.async_copy(g_hbm.at[src_v.at[j + 2]], rows.at[p], gsem[p])

    def run(src_arr, dst_arr, nb, nsr):
        nch = (nb + _CD - 1) // _CD
        pltpu.sync_copy(src_arr.at[s], src_v.at[pl.ds(0, nsr)])
        pltpu.sync_copy(dst_arr.at[s, pl.ds(0, _CD)], dstb.at[0])
        if nch > 1:
            pltpu.async_copy(dst_arr.at[s, pl.ds(_CD, _CD)], dstb.at[1], csem1)
        pltpu.async_copy(g_hbm.at[src_v.at[0]], rows.at[0], gsem0)
        pltpu.async_copy(g_hbm.at[src_v.at[1]], rows.at[1], gsem1)

        for ch in range(nch):
            q = ch % 2
            base = ch * _CD
            last = ch + 1 == nch
            if ch > 0:
                pltpu.make_async_copy(
                    dst_arr.at[s, pl.ds(base, _CD)], dstb.at[q], csem[q]).wait()
                if not last:
                    pltpu.async_copy(
                        dst_arr.at[s, pl.ds(base + _CD, _CD)],
                        dstb.at[1 - q], csem[1 - q])

            rb = min(_CD, nb - base)
            if not last:
                def body(t, carry, base=base, q=q):
                    consume(base + 2 * t, 0, q, 2 * t, True)
                    consume(base + 2 * t + 1, 1, q, 2 * t + 1, True)
                    return carry
                lax.fori_loop(0, _CD // 2, body, 0)
            else:
                def tbody(t, carry, base=base, q=q):
                    consume(base + 2 * t, 0, q, 2 * t, True)
                    consume(base + 2 * t + 1, 1, q, 2 * t + 1, True)
                    return carry
                lax.fori_loop(0, (rb - 2) // 2, tbody, 0)
                consume(base + rb - 2, 0, q, rb - 2, False)
                consume(base + rb - 1, 1, q, rb - 1, False)

    @pl.when(c == 0)
    def _():
        run(srcA, dstA, _NB0, _rup8(_NB0))

    @pl.when(c == 1)
    def _():
        run(srcB, dstB, _NB1, _rup8(_NB1))

    plsc.subcore_barrier()
    pltpu.sync_copy(acc_sp.at[pl.ds(r0, _RPT)], out_hbm.at[c, pl.ds(r0, _RPT)])


def _gemm_block(x_ref, w_ref, dp_ref, g_ref):
    deg = dp_ref[0, :, 0] + dp_ref[1, :, 0] + 1.0
    dis = lax.rsqrt(deg)
    h = jnp.dot(x_ref[...], w_ref[...], preferred_element_type=jnp.float32)
    g_ref[...] = h * dis[:, None]


def _combine_block(p_ref, g_ref, dp_ref, b_ref, o_ref):
    deg = dp_ref[0, :, 0] + dp_ref[1, :, 0] + 1.0
    dis = lax.rsqrt(deg)
    t = (p_ref[0] + p_ref[1] + g_ref[...]) * dis[:, None] + b_ref[...]
    o_ref[...] = jnp.maximum(t, 0.0)


_RB = 1024


def kernel(x, edge_index, W, b):
    src = edge_index[0]
    dst = edge_index[1]
    pad = _EPAD - _E
    src_p = jnp.concatenate([src, jnp.zeros((pad,), jnp.int32)])
    dst_p = jnp.concatenate([dst, jnp.full((pad,), _N, jnp.int32)])
    dst_r = dst_p.reshape(_NW, _NB, _K)
    sp2 = src_p.reshape(_NS, 2 * _EPW)
    dp2 = dst_p.reshape(_NS, 2 * _EPW)
    cut = _NB0 * _K
    padA = _rup8(_NB0) - _NB0
    padB = _rup8(_NB1) - _NB1
    srcA = jnp.concatenate(
        [sp2[:, :cut].reshape(_NS, _NB0, _K),
         jnp.zeros((_NS, padA, _K), jnp.int32)], axis=1)
    dstA = jnp.concatenate(
        [dp2[:, :cut].reshape(_NS, _NB0, _K),
         jnp.full((_NS, padA, _K), _N, jnp.int32)], axis=1)
    srcB = jnp.concatenate(
        [sp2[:, cut:].reshape(_NS, _NB1, _K),
         jnp.zeros((_NS, padB, _K), jnp.int32)], axis=1)
    dstB = jnp.concatenate(
        [dp2[:, cut:].reshape(_NS, _NB1, _K),
         jnp.full((_NS, padB, _K), _N, jnp.int32)], axis=1)

    zerosd = jnp.zeros((_N1, _D), jnp.float32)
    onesd = jnp.ones((_K, _D), jnp.float32)

    deg_parts = _deg_kernel(dst_r, onesd, zerosd)

    g = pl.pallas_call(
        _gemm_block,
        grid=((_N + _RB - 1) // _RB,),
        in_specs=[
            pl.BlockSpec((_RB, _D), lambda i: (i, 0)),
            pl.BlockSpec((_D, _D), lambda i: (0, 0)),
            pl.BlockSpec((_NC, _RB, _D), lambda i: (0, i, 0)),
        ],
        out_specs=pl.BlockSpec((_RB, _D), lambda i: (i, 0)),
        out_shape=jax.ShapeDtypeStruct((_N, _D), jnp.float32),
    )(x, W, deg_parts)

    acc_parts = _scatter_kernel(srcA, dstA, srcB, dstB, g, zerosd)

    out = pl.pallas_call(
        _combine_block,
        grid=((_N + _RB - 1) // _RB,),
        in_specs=[
            pl.BlockSpec((_NC, _RB, _D), lambda i: (0, i, 0)),
            pl.BlockSpec((_RB, _D), lambda i: (i, 0)),
            pl.BlockSpec((_NC, _RB, _D), lambda i: (0, i, 0)),
            pl.BlockSpec((1, _D), lambda i: (0, 0)),
        ],
        out_specs=pl.BlockSpec((_RB, _D), lambda i: (i, 0)),
        out_shape=jax.ShapeDtypeStruct((_N, _D), jnp.float32),
    )(acc_parts, g, deg_parts, b.reshape(1, _D))

    return out


# speedup vs baseline: 1.1581x; 1.1581x over previous
import functools

import jax
import jax.numpy as jnp
from jax import lax
from jax.experimental import pallas as pl
from jax.experimental.pallas import tpu as pltpu
from jax.experimental.pallas import tpu_sc as plsc

_N = 10000
_D = 128
_E = 320000

_NC = 2
_NS = 16
_NW = _NC * _NS

_K = 128
_NB = 79
_EPW = _K * _NB
_EPAD = 2 * _EPW * _NS

_NB0 = 56
_NB1 = 2 * _NB - _NB0
_CD = 8

_RPT = 632
_N1 = _RPT * _NS

_mesh = plsc.VectorSubcoreMesh(core_axis_name="c", subcore_axis_name="s")


@functools.partial(
    pl.kernel,
    out_type=jax.ShapeDtypeStruct((_NC, _N1, _D), jnp.float32),
    mesh=_mesh,
    scratch_types=[
        pltpu.VMEM((_NB, _K), jnp.int32),
        pltpu.VMEM((_K, _D), jnp.float32),
        pltpu.VMEM_SHARED((_N1, _D), jnp.float32),
    ],
)
def _deg_kernel(dst_hbm, ones_hbm, zeros_hbm, out_hbm, dst_v, ones_v, deg_sp):
    c = lax.axis_index("c")
    s = lax.axis_index("s")
    wid = s * _NC + c
    r0 = s * _RPT
    pltpu.sync_copy(zeros_hbm.at[pl.ds(r0, _RPT)], deg_sp.at[pl.ds(r0, _RPT)])
    pltpu.sync_copy(ones_hbm, ones_v)
    pltpu.sync_copy(dst_hbm.at[wid], dst_v)
    plsc.subcore_barrier()

    def body(j, carry):
        pltpu.sync_copy(ones_v, deg_sp.at[dst_v.at[j]], add=True)
        return carry

    lax.fori_loop(0, _NB, body, 0)
    plsc.subcore_barrier()
    pltpu.sync_copy(deg_sp.at[pl.ds(r0, _RPT)], out_hbm.at[c, pl.ds(r0, _RPT)])


@functools.partial(
    pl.kernel,
    out_type=jax.ShapeDtypeStruct((_NC, _N1, _D), jnp.float32),
    mesh=_mesh,
    scratch_types=[
        pltpu.VMEM((104, _K), jnp.int32),
        pltpu.VMEM((2, _CD, _K), jnp.int32),
        pltpu.VMEM((2, _D, _D), jnp.float32),
        pltpu.VMEM_SHARED((_N1, _D), jnp.float32),
        pltpu.SemaphoreType.DMA,
        pltpu.SemaphoreType.DMA,
        pltpu.SemaphoreType.DMA,
        pltpu.SemaphoreType.DMA,
        pltpu.SemaphoreType.DMA,
        pltpu.SemaphoreType.DMA,
    ],
)
def _scatter_kernel(srcA, dstA, srcB, dstB, g_hbm, zeros_hbm, out_hbm,
                    src_v, dstb, rows, acc_sp,
                    gsem0, gsem1, ssem0, ssem1, csem0, csem1):
    c = lax.axis_index("c")
    s = lax.axis_index("s")
    r0 = s * _RPT
    pltpu.sync_copy(zeros_hbm.at[pl.ds(r0, _RPT)], acc_sp.at[pl.ds(r0, _RPT)])
    plsc.subcore_barrier()

    gsem = (gsem0, gsem1)
    ssem = (ssem0, ssem1)
    csem = (csem0, csem1)

    def consume(j, p, q, i, gather_next):
        pltpu.make_async_copy(g_hbm.at[src_v.at[j]], rows.at[p], gsem[p]).wait()
        pltpu.async_copy(rows.at[p], acc_sp.at[dstb.at[q, i]], ssem[p], add=True)
        pltpu.make_async_copy(rows.at[p], acc_sp.at[dstb.at[q, i]], ssem[p]).wait()
        if gather_next:
            pltpu.async_copy(g_hbm.at[src_v.at[j + 2]], rows.at[p], gsem[p])

    def run(src_arr, dst_arr, nb, nsr):
        nch = (nb + _CD - 1) // _CD
        pltpu.sync_copy(src_arr.at[s], src_v.at[pl.ds(0, nsr)])
        pltpu.sync_copy(dst_arr.at[s, pl.ds(0, _CD)], dstb.at[0])
        pltpu.async_copy(dst_arr.at[s, pl.ds(_CD, _CD)], dstb.at[1], csem1)
        pltpu.async_copy(g_hbm.at[src_v.at[0]], rows.at[0], gsem0)
        pltpu.async_copy(g_hbm.at[src_v.at[1]], rows.at[1], gsem1)

        for ch in range(nch):
            q = ch % 2
            base = ch * _CD
            if ch > 0:
                pltpu.make_async_copy(
                    dst_arr.at[s, pl.ds(base, _CD)], dstb.at[q], csem[q]).wait()
                if ch + 1 < nch:
                    pltpu.async_copy(
                        dst_arr.at[s, pl.ds(base + _CD, _CD)],
                        dstb.at[1 - q], csem[1 - q])

            rb = min(_CD, nb - base)
            if ch < nch - 1:
                def body(t, carry, base=base, q=q):
                    consume(base + 2 * t, 0, q, 2 * t, True)
                    consume(base + 2 * t + 1, 1, q, 2 * t + 1, True)
                    return carry
                lax.fori_loop(0, _CD // 2, body, 0)
            else:
                def tbody(t, carry, base=base, q=q):
                    consume(base + 2 * t, 0, q, 2 * t, True)
                    consume(base + 2 * t + 1, 1, q, 2 * t + 1, True)
                    return carry
                lax.fori_loop(0, (rb - 2) // 2, tbody, 0)
                consume(base + rb - 2, 0, q, rb - 2, False)
                consume(base + rb - 1, 1, q, rb - 1, False)

    @pl.when(c == 0)
    def _():
        run(srcA, dstA, _NB0, _NB0)

    @pl.when(c == 1)
    def _():
        run(srcB, dstB, _NB1, 104)

    plsc.subcore_barrier()
    pltpu.sync_copy(acc_sp.at[pl.ds(r0, _RPT)], out_hbm.at[c, pl.ds(r0, _RPT)])


def _gemm_block(x_ref, w_ref, dp_ref, g_ref):
    deg = dp_ref[0, :, 0] + dp_ref[1, :, 0] + 1.0
    dis = lax.rsqrt(deg)
    h = jnp.dot(x_ref[...], w_ref[...], preferred_element_type=jnp.float32)
    g_ref[...] = h * dis[:, None]


def _combine_block(p_ref, g_ref, dp_ref, b_ref, o_ref):
    deg = dp_ref[0, :, 0] + dp_ref[1, :, 0] + 1.0
    dis = lax.rsqrt(deg)
    t = (p_ref[0] + p_ref[1] + g_ref[...]) * dis[:, None] + b_ref[...]
    o_ref[...] = jnp.maximum(t, 0.0)


_RB = 1000


def kernel(x, edge_index, W, b):
    src = edge_index[0]
    dst = edge_index[1]
    pad = _EPAD - _E
    src_p = jnp.concatenate([src, jnp.zeros((pad,), jnp.int32)])
    dst_p = jnp.concatenate([dst, jnp.full((pad,), _N, jnp.int32)])
    dst_r = dst_p.reshape(_NW, _NB, _K)
    sp2 = src_p.reshape(_NS, 2 * _EPW)
    dp2 = dst_p.reshape(_NS, 2 * _EPW)
    cut = _NB0 * _K
    srcA = sp2[:, :cut].reshape(_NS, _NB0, _K)
    dstA = dp2[:, :cut].reshape(_NS, _NB0, _K)
    srcB = jnp.concatenate(
        [sp2[:, cut:].reshape(_NS, _NB1, _K),
         jnp.zeros((_NS, 2, _K), jnp.int32)], axis=1)
    dstB = jnp.concatenate(
        [dp2[:, cut:].reshape(_NS, _NB1, _K),
         jnp.full((_NS, 2, _K), _N, jnp.int32)], axis=1)

    onesd = jnp.ones((_K, _D), jnp.float32)
    zerosd = jnp.zeros((_N1, _D), jnp.float32)

    deg_parts = _deg_kernel(dst_r, onesd, zerosd)

    g = pl.pallas_call(
        _gemm_block,
        grid=(_N // _RB,),
        in_specs=[
            pl.BlockSpec((_RB, _D), lambda i: (i, 0)),
            pl.BlockSpec((_D, _D), lambda i: (0, 0)),
            pl.BlockSpec((_NC, _RB, _D), lambda i: (0, i, 0)),
        ],
        out_specs=pl.BlockSpec((_RB, _D), lambda i: (i, 0)),
        out_shape=jax.ShapeDtypeStruct((_N, _D), jnp.float32),
    )(x, W, deg_parts)

    acc_parts = _scatter_kernel(srcA, dstA, srcB, dstB, g, zerosd)

    out = pl.pallas_call(
        _combine_block,
        grid=(_N // _RB,),
        in_specs=[
            pl.BlockSpec((_NC, _RB, _D), lambda i: (0, i, 0)),
            pl.BlockSpec((_RB, _D), lambda i: (i, 0)),
            pl.BlockSpec((_NC, _RB, _D), lambda i: (0, i, 0)),
            pl.BlockSpec((1, _D), lambda i: (0, 0)),
        ],
        out_specs=pl.BlockSpec((_RB, _D), lambda i: (i, 0)),
        out_shape=jax.ShapeDtypeStruct((_N, _D), jnp.float32),
    )(acc_parts, g, deg_parts, b.reshape(1, _D))

    return out
